# Initial kernel scaffold; baseline (speedup 1.0000x reference)
import jax
import jax.numpy as jnp
from jax.experimental import pallas as pl


def kernel(relation_representations, rpW1, rpb1, lW1, lb1, lng1, lnb1, rpW2, rpb2, lW2, lb2, lng2, lnb2, h_index, r_index, edge_index, edge_type):
    raise NotImplementedError("write your pallas kernel here")


# speedup vs baseline: 21.3069x; 21.3069x over previous
import functools

import jax
import jax.numpy as jnp
from jax import lax
from jax.experimental import pallas as pl
from jax.experimental.pallas import tpu as pltpu
from jax.experimental.pallas import tpu_sc as plsc

N = 10000
E = 160000
D = 128
R = 16
B = 2

NTILE = 16
N_PAD = 10240
ROWS_PT = N_PAD // NTILE
CH = 80
E_PAD = 161280
EPT = E_PAD // NTILE
NCHUNK = EPT // CH
BN = 400
NB = N // BN


def _sc_msg_pass(gather_x: bool, x_rows: int):
  mesh = plsc.VectorSubcoreMesh(core_axis_name="c", subcore_axis_name="s")

  def body(src_hbm, dst_hbm, et_hbm, tab_hbm, x_hbm, out_hbm,
           agg_sh, src_v, dst_v, et_v, rows_v, tab_v, zb_v, gsem):
    c = lax.axis_index("c")
    s = lax.axis_index("s")
    b = c
    lane = lax.iota(jnp.int32, 16)

    pltpu.sync_copy(tab_hbm.at[pl.ds(b * 24, 24)], tab_v)

    def zb_zero(i, carry):
      for j in range(D // 16):
        zb_v[i, pl.ds(j * 16, 16)] = jnp.zeros((16,), jnp.float32)
      return carry
    lax.fori_loop(0, CH, zb_zero, 0)
    for m in range(ROWS_PT // CH):
      pltpu.sync_copy(zb_v, agg_sh.at[pl.ds(s * ROWS_PT + m * CH, CH)])
    plsc.subcore_barrier()

    def chunk_body(k, carry):
      base = b * E_PAD + s * EPT + k * CH
      pltpu.sync_copy(dst_hbm.at[pl.ds(base, CH)], dst_v)
      pltpu.sync_copy(et_hbm.at[pl.ds(base, CH)], et_v.at[pl.ds(0, CH)])
      if gather_x:
        pltpu.sync_copy(src_hbm.at[pl.ds(base, CH)], src_v)
        pltpu.async_copy(x_hbm.at[src_v], rows_v, gsem).wait()

      def edge_body(i, carry2):
        et_i = et_v[pl.ds(i, 16)][0]
        for j in range(D // 16):
          rv = tab_v[et_i, pl.ds(j * 16, 16)]
          if gather_x:
            rv = rows_v[i, pl.ds(j * 16, 16)] * rv
          rows_v[i, pl.ds(j * 16, 16)] = rv
        return carry2
      lax.fori_loop(0, CH, edge_body, 0)

      pltpu.sync_copy(rows_v, agg_sh.at[dst_v], add=True)
      return carry
    lax.fori_loop(0, NCHUNK, chunk_body, 0)
    plsc.subcore_barrier()

    pltpu.sync_copy(agg_sh.at[pl.ds(s * ROWS_PT, ROWS_PT)],
                    out_hbm.at[pl.ds(b * N_PAD + s * ROWS_PT, ROWS_PT)])

  kern = pl.kernel(
      body,
      out_type=jax.ShapeDtypeStruct((B * N_PAD, D), jnp.float32),
      mesh=mesh,
      scratch_types=[
          pltpu.VMEM_SHARED((N_PAD, D), jnp.float32),
          pltpu.VMEM((CH,), jnp.int32),
          pltpu.VMEM((CH,), jnp.int32),
          pltpu.VMEM((CH + 16,), jnp.int32),
          pltpu.VMEM((CH, D), jnp.float32),
          pltpu.VMEM((24, D), jnp.float32),
          pltpu.VMEM((CH, D), jnp.float32),
          pltpu.SemaphoreType.DMA,
      ],
  )

  def call(src, dst, et, tab, x):
    if not gather_x:
      x = jnp.zeros((8, D), jnp.float32)
    tab24 = jnp.concatenate(
        [tab, jnp.zeros((B, 24 - tab.shape[1], D), jnp.float32)], axis=1)
    return kern(src.reshape(-1), dst.reshape(-1), et.reshape(-1),
                tab24.reshape(B * 24, D), x)
  return call


_sc_pass_nogather = _sc_msg_pass(False, 8)
_sc_pass_gather = _sc_msg_pass(True, B * N + 16)


def _k0_body(rel_ref, w1_ref, b1_ref, w2_ref, b2_ref, q_ref,
             qrel1_ref, rel2_ref):
  ra = rel_ref[...]
  dn = (((1,), (1,)), ((), ()))
  r1 = lax.dot_general(ra, w1_ref[...], dn) + b1_ref[...]
  r2 = lax.dot_general(ra, w2_ref[...], dn) + b2_ref[...]
  q = q_ref[...]
  qb = jnp.broadcast_to(q[:, None, :], (B, R, D)).reshape(B * R, D)
  qrel1_ref[...] = r1 * qb
  rel2_ref[...] = r2


def _k0(rel_flat, rpW1, rpb1, rpW2, rpb2, query):
  return pl.pallas_call(
      _k0_body,
      out_shape=[
          jax.ShapeDtypeStruct((B * R, D), jnp.float32),
          jax.ShapeDtypeStruct((B * R, D), jnp.float32),
      ],
  )(rel_flat, rpW1, rpb1, rpW2, rpb2, query)


def _dense_body(final, x_ref, a_ref, w1_ref, w2_ref, lb_ref, lng_ref,
                lnb_ref, q_ref, o_ref):
  x = x_ref[0]
  a = a_ref[0]
  dn = (((1,), (1,)), ((), ()))
  t = (lax.dot_general(x, w1_ref[...], dn)
       + lax.dot_general(a, w2_ref[...], dn) + lb_ref[...])
  mu = jnp.mean(t, axis=-1, keepdims=True)
  d = t - mu
  var = jnp.mean(d * d, axis=-1, keepdims=True)
  y = d * lax.rsqrt(var + 1e-5) * lng_ref[...] + lnb_ref[...]
  y = jnp.maximum(y, 0.0) + x
  if final:
    qb = jnp.broadcast_to(q_ref[0, 0:1, :], (BN, D))
    o_ref[0] = jnp.concatenate([y, qb], axis=-1)
  else:
    o_ref[0] = y


def _dense(final, x, agg_pad, w1, w2, lb, lng, lnb, query):
  od = 2 * D if final else D
  full = lambda shape: pl.BlockSpec(shape, lambda bb, nb: (0, 0))
  return pl.pallas_call(
      functools.partial(_dense_body, final),
      grid=(B, NB),
      in_specs=[
          pl.BlockSpec((1, BN, D), lambda bb, nb: (bb, nb, 0)),
          pl.BlockSpec((1, BN, D), lambda bb, nb: (bb, nb, 0)),
          full((D, D)),
          full((D, D)),
          full((1, D)),
          full((1, D)),
          full((1, D)),
          pl.BlockSpec((1, 8, D), lambda bb, nb: (bb, 0, 0)),
      ],
      out_specs=pl.BlockSpec((1, BN, od), lambda bb, nb: (bb, nb, 0)),
      out_shape=jax.ShapeDtypeStruct((B, N, od), jnp.float32),
  )(x, agg_pad, w1, w2, lb, lng, lnb,
    jnp.broadcast_to(query[:, None, :], (B, 8, D)))


def kernel(relation_representations, rpW1, rpb1, lW1, lb1, lng1, lnb1,
           rpW2, rpb2, lW2, lb2, lng2, lnb2,
           h_index, r_index, edge_index, edge_type):
  rel = relation_representations.astype(jnp.float32)
  h_index = h_index.astype(jnp.int32)
  r_index = r_index.astype(jnp.int32)
  src = edge_index[0].astype(jnp.int32)
  dst = edge_index[1].astype(jnp.int32)
  et = edge_type.astype(jnp.int32)

  query = jnp.take_along_axis(rel, r_index[:, None, None], axis=1)[:, 0, :]

  qrel1, rel2p = _k0(rel.reshape(B * R, D), rpW1, rpb1.reshape(1, D),
                     rpW2, rpb2.reshape(1, D), query)
  tab1 = jnp.concatenate(
      [qrel1.reshape(B, R, D), query[:, None, :]], axis=1)
  tab2 = jnp.concatenate(
      [rel2p.reshape(B, R, D), jnp.ones((B, 1, D), jnp.float32)], axis=1)

  npad = E_PAD - E - 1
  iz = jnp.zeros((npad,), jnp.int32)
  dump = jnp.full((npad,), N, jnp.int32)
  et_b = jnp.broadcast_to(et[None], (B, E))
  etA = jnp.concatenate(
      [et_b, jnp.full((B, 1), R, jnp.int32),
       jnp.broadcast_to(iz[None], (B, npad))], axis=1)
  dst_pad = jnp.concatenate(
      [jnp.broadcast_to(dst[None], (B, E)), h_index[:, None],
       jnp.broadcast_to(dump[None], (B, npad))], axis=1)
  m1 = src[None, :] == h_index[:, None]
  dstA = jnp.concatenate(
      [jnp.where(m1, dst[None, :], N), h_index[:, None],
       jnp.broadcast_to(dump[None], (B, npad))], axis=1)
  boff = jnp.arange(B, dtype=jnp.int32)[:, None] * N
  srcB = jnp.concatenate(
      [src[None, :] + boff, B * N + 8 * jnp.arange(B, dtype=jnp.int32)[:, None],
       jnp.broadcast_to(iz[None], (B, npad))], axis=1)

  agg1 = _sc_pass_nogather(srcB, dstA, etA, tab1, None)
  agg1 = agg1.reshape(B, N_PAD, D)
  hoh = (jnp.arange(N, dtype=jnp.int32)[None, :] == h_index[:, None])
  x0 = jnp.where(hoh[:, :, None], query[:, None, :], 0.0)

  h1 = _dense(False, x0, agg1, lW1[:, :D], lW1[:, D:],
              lb1.reshape(1, D), lng1.reshape(1, D), lnb1.reshape(1, D),
              query)

  qpad = jnp.zeros((16, D), jnp.float32).at[jnp.arange(B) * 8].set(query)
  x1 = jnp.concatenate([h1.reshape(B * N, D), qpad], axis=0)
  agg2 = _sc_pass_gather(srcB, dst_pad, etA, tab2, x1)
  agg2 = agg2.reshape(B, N_PAD, D)

  return _dense(True, h1, agg2, lW2[:, :D], lW2[:, D:],
                lb2.reshape(1, D), lng2.reshape(1, D), lnb2.reshape(1, D),
                query)


# speedup vs baseline: 32.1826x; 1.5104x over previous
import functools

import jax
import jax.numpy as jnp
from jax import lax
from jax.experimental import pallas as pl
from jax.experimental.pallas import tpu as pltpu
from jax.experimental.pallas import tpu_sc as plsc

N = 10000
E = 160000
D = 128
R = 16
B = 2

NTILE = 16
N_PAD = 10240
ROWS_PT = N_PAD // NTILE
CH = 128
E_PAD = 163840
EPT = E_PAD // NTILE
NCHUNK = EPT // CH
BN = 400
NB = N // BN


def _sc_msg_pass(hist: bool):
  W = D
  mesh = plsc.VectorSubcoreMesh(core_axis_name="c", subcore_axis_name="s")

  def body(src_hbm, dst_hbm, et_hbm, tab_hbm, x_hbm, out_hbm,
           agg_sh, src_b, dst_b, et_b, rows_b, tab_v,
           sem_i, sem_g):
    c = lax.axis_index("c")
    s = lax.axis_index("s")
    b = c
    lane = lax.iota(jnp.int32, 16)
    zero16 = jnp.zeros((16,), jnp.float32)

    if not hist:
      pltpu.sync_copy(tab_hbm.at[pl.ds(b * 24, 24)], tab_v)

    def rz(i, carry):
      for j in range(W // 16):
        rows_b[0][i, pl.ds(j * 16, 16)] = zero16
        if hist:
          rows_b[1][i, pl.ds(j * 16, 16)] = zero16
      return carry
    lax.fori_loop(0, CH, rz, 0)
    for m in range(ROWS_PT // CH):
      pltpu.sync_copy(rows_b[0], agg_sh.at[pl.ds(s * ROWS_PT + m * CH, CH)])

    def issue_idx(k, p):
      base = b * E_PAD + s * EPT + k * CH
      if not hist:
        pltpu.async_copy(src_hbm.at[pl.ds(base, CH)], src_b[p], sem_i[p])
      pltpu.async_copy(dst_hbm.at[pl.ds(base, CH)], dst_b[p], sem_i[p])
      pltpu.async_copy(et_hbm.at[pl.ds(base, CH)], et_b[p], sem_i[p])

    def wait_idx(k, p):
      base = b * E_PAD + s * EPT + k * CH
      if not hist:
        pltpu.make_async_copy(
            src_hbm.at[pl.ds(base, CH)], src_b[p], sem_i[p]).wait()
      pltpu.make_async_copy(
          dst_hbm.at[pl.ds(base, CH)], dst_b[p], sem_i[p]).wait()
      pltpu.make_async_copy(
          et_hbm.at[pl.ds(base, CH)], et_b[p], sem_i[p]).wait()

    issue_idx(0, 0)
    issue_idx(1, 1)
    if not hist:
      wait_idx(0, 0)
      pltpu.async_copy(x_hbm.at[src_b[0]], rows_b[0], sem_g[0])
    plsc.subcore_barrier()

    def compute_chunk(p):
      def group_body(g, carry):
        ets16 = et_b[p][pl.ds(g * 16, 16)]
        for ii in range(16):
          et_i = ets16[ii]
          i = g * 16 + ii
          if hist:
            rows_b[p][i, pl.ds(0, 16)] = jnp.where(
                lane == et_i, jnp.float32(1.0), jnp.float32(0.0))
          else:
            for j in range(D // 16):
              rv = tab_v[et_i, pl.ds(j * 16, 16)]
              rows_b[p][i, pl.ds(j * 16, 16)] = (
                  rows_b[p][i, pl.ds(j * 16, 16)] * rv)
        return carry
      lax.fori_loop(0, CH // 16, group_body, 0)

    def chunk_pair(k2, carry):
      for p in (0, 1):
        q = 1 - p
        k = 2 * k2 + p
        if hist:
          wait_idx(k, p)
        else:
          pltpu.make_async_copy(
              x_hbm.at[src_b[p]], rows_b[p], sem_g[p]).wait()
        compute_chunk(p)
        pltpu.sync_copy(rows_b[p], agg_sh.at[dst_b[p]], add=True)
        if not hist:
          @pl.when(k + 1 < NCHUNK)
          def _():
            wait_idx(k + 1, q)
            pltpu.async_copy(x_hbm.at[src_b[q]], rows_b[q], sem_g[q])

        @pl.when(k + 2 < NCHUNK)
        def _():
          issue_idx(k + 2, p)
      return carry
    lax.fori_loop(0, NCHUNK // 2, chunk_pair, 0)
    plsc.subcore_barrier()

    pltpu.sync_copy(agg_sh.at[pl.ds(s * ROWS_PT, ROWS_PT)],
                    out_hbm.at[pl.ds(b * N_PAD + s * ROWS_PT, ROWS_PT)])

  kern = pl.kernel(
      body,
      out_type=jax.ShapeDtypeStruct((B * N_PAD, W), jnp.float32),
      mesh=mesh,
      scratch_types=[
          pltpu.VMEM_SHARED((N_PAD, W), jnp.float32),
          [pltpu.VMEM((CH,), jnp.int32), pltpu.VMEM((CH,), jnp.int32)],
          [pltpu.VMEM((CH,), jnp.int32), pltpu.VMEM((CH,), jnp.int32)],
          [pltpu.VMEM((CH,), jnp.int32), pltpu.VMEM((CH,), jnp.int32)],
          [pltpu.VMEM((CH, W), jnp.float32), pltpu.VMEM((CH, W), jnp.float32)],
          pltpu.VMEM((24, D), jnp.float32),
          [pltpu.SemaphoreType.DMA, pltpu.SemaphoreType.DMA],
          [pltpu.SemaphoreType.DMA, pltpu.SemaphoreType.DMA],
      ],
  )

  def call(src, dst, et, tab, x):
    if hist:
      x = jnp.zeros((8, D), jnp.float32)
      tab24 = jnp.zeros((B * 24, D), jnp.float32)
    else:
      tab24 = jnp.concatenate(
          [tab, jnp.zeros((B, 24 - tab.shape[1], D), jnp.float32)],
          axis=1).reshape(B * 24, D)
    return kern(src.reshape(-1), dst.reshape(-1), et.reshape(-1), tab24, x)
  return call


_sc_pass_hist = _sc_msg_pass(True)
_sc_pass_gather = _sc_msg_pass(False)


def _k0_body(rel_ref, w1_ref, b1_ref, w2_ref, b2_ref, q_ref,
             qrel1_ref, rel2_ref):
  ra = rel_ref[...]
  dn = (((1,), (1,)), ((), ()))
  r1 = lax.dot_general(ra, w1_ref[...], dn) + b1_ref[...]
  r2 = lax.dot_general(ra, w2_ref[...], dn) + b2_ref[...]
  q = q_ref[...]
  qb = jnp.broadcast_to(q[:, None, :], (B, R, D)).reshape(B * R, D)
  qrel1_ref[...] = r1 * qb
  rel2_ref[...] = r2


def _k0(rel_flat, rpW1, rpb1, rpW2, rpb2, query):
  return pl.pallas_call(
      _k0_body,
      out_shape=[
          jax.ShapeDtypeStruct((B * R, D), jnp.float32),
          jax.ShapeDtypeStruct((B * R, D), jnp.float32),
      ],
  )(rel_flat, rpW1, rpb1, rpW2, rpb2, query)


def _dense_body(final, hist, x_ref, a_ref, tab_ref, w1_ref, w2_ref, lb_ref,
                lng_ref, lnb_ref, q_ref, o_ref):
  x = x_ref[0]
  dn = (((1,), (1,)), ((), ()))
  if hist:
    cnt = a_ref[0][:, :16]
    a = lax.dot_general(cnt, tab_ref[0], (((1,), (0,)), ((), ()))) + x
  else:
    a = a_ref[0]
  t = (lax.dot_general(x, w1_ref[...], dn)
       + lax.dot_general(a, w2_ref[...], dn) + lb_ref[...])
  mu = jnp.mean(t, axis=-1, keepdims=True)
  d = t - mu
  var = jnp.mean(d * d, axis=-1, keepdims=True)
  y = d * lax.rsqrt(var + 1e-5) * lng_ref[...] + lnb_ref[...]
  y = jnp.maximum(y, 0.0) + x
  if final:
    qb = jnp.broadcast_to(q_ref[0, 0:1, :], (BN, D))
    o_ref[0] = jnp.concatenate([y, qb], axis=-1)
  else:
    o_ref[0] = y


def _dense(final, hist, x, agg_pad, tab, w1, w2, lb, lng, lnb, query):
  od = 2 * D if final else D
  aw = D
  full = lambda shape: pl.BlockSpec(shape, lambda bb, nb: (0, 0))
  if tab is None:
    tab = jnp.zeros((B, 1, 8, D), jnp.float32)
  return pl.pallas_call(
      functools.partial(_dense_body, final, hist),
      grid=(B, NB),
      in_specs=[
          pl.BlockSpec((1, BN, D), lambda bb, nb: (bb, nb, 0)),
          pl.BlockSpec((1, BN, aw), lambda bb, nb: (bb, nb, 0)),
          pl.BlockSpec((1, 16, D), lambda bb, nb: (bb, 0, 0)),
          full((D, D)),
          full((D, D)),
          full((1, D)),
          full((1, D)),
          full((1, D)),
          pl.BlockSpec((1, 8, D), lambda bb, nb: (bb, 0, 0)),
      ],
      out_specs=pl.BlockSpec((1, BN, od), lambda bb, nb: (bb, nb, 0)),
      out_shape=jax.ShapeDtypeStruct((B, N, od), jnp.float32),
  )(x, agg_pad, tab.reshape(B, 16, D) if hist else jnp.zeros((B, 16, D), jnp.float32),
    w1, w2, lb, lng, lnb,
    jnp.broadcast_to(query[:, None, :], (B, 8, D)))


def kernel(relation_representations, rpW1, rpb1, lW1, lb1, lng1, lnb1,
           rpW2, rpb2, lW2, lb2, lng2, lnb2,
           h_index, r_index, edge_index, edge_type):
  rel = relation_representations.astype(jnp.float32)
  h_index = h_index.astype(jnp.int32)
  r_index = r_index.astype(jnp.int32)
  src = edge_index[0].astype(jnp.int32)
  dst = edge_index[1].astype(jnp.int32)
  et = edge_type.astype(jnp.int32)

  query = jnp.take_along_axis(rel, r_index[:, None, None], axis=1)[:, 0, :]

  qrel1, rel2p = _k0(rel.reshape(B * R, D), rpW1, rpb1.reshape(1, D),
                     rpW2, rpb2.reshape(1, D), query)
  tab2 = jnp.concatenate(
      [rel2p.reshape(B, R, D), jnp.ones((B, 1, D), jnp.float32)], axis=1)

  npad = E_PAD - E - 1
  iz = jnp.zeros((npad,), jnp.int32)
  dump = jnp.full((npad,), N, jnp.int32)
  et_b = jnp.broadcast_to(et[None], (B, E))
  etA = jnp.concatenate(
      [et_b, jnp.full((B, 1), R, jnp.int32),
       jnp.broadcast_to(iz[None], (B, npad))], axis=1)
  dst_pad = jnp.concatenate(
      [jnp.broadcast_to(dst[None], (B, E)), h_index[:, None],
       jnp.broadcast_to(dump[None], (B, npad))], axis=1)
  m1 = src[None, :] == h_index[:, None]
  dstA = jnp.concatenate(
      [jnp.where(m1, dst[None, :], N), jnp.full((B, 1), N, jnp.int32),
       jnp.broadcast_to(dump[None], (B, npad))], axis=1)
  boff = jnp.arange(B, dtype=jnp.int32)[:, None] * N
  srcB = jnp.concatenate(
      [src[None, :] + boff,
       B * N + 8 * jnp.arange(B, dtype=jnp.int32)[:, None],
       jnp.broadcast_to(iz[None], (B, npad))], axis=1)

  cnt1 = _sc_pass_hist(srcB, dstA, etA, None, None)
  cnt1 = cnt1.reshape(B, N_PAD, D)
  hoh = (jnp.arange(N, dtype=jnp.int32)[None, :] == h_index[:, None])
  x0 = jnp.where(hoh[:, :, None], query[:, None, :], 0.0)

  h1 = _dense(False, True, x0, cnt1, qrel1, lW1[:, :D], lW1[:, D:],
              lb1.reshape(1, D), lng1.reshape(1, D), lnb1.reshape(1, D),
              query)

  qpad = jnp.zeros((16, D), jnp.float32).at[jnp.arange(B) * 8].set(query)
  x1 = jnp.concatenate([h1.reshape(B * N, D), qpad], axis=0)
  agg2 = _sc_pass_gather(srcB, dst_pad, etA, tab2, x1)
  agg2 = agg2.reshape(B, N_PAD, D)

  return _dense(True, False, h1, agg2, None, lW2[:, :D], lW2[:, D:],
                lb2.reshape(1, D), lng2.reshape(1, D), lnb2.reshape(1, D),
                query)


# speedup vs baseline: 37.6923x; 1.1712x over previous
import functools

import jax
import jax.numpy as jnp
from jax import lax
from jax.experimental import pallas as pl
from jax.experimental.pallas import tpu as pltpu
from jax.experimental.pallas import tpu_sc as plsc

N = 10000
E = 160000
D = 128
R = 16
B = 2

NTILE = 16
N_PAD = 10240
ROWS_PT = N_PAD // NTILE
CH = 80
E_PAD = 163840
EPT = E_PAD // NTILE
NCHUNK = EPT // CH
NBUF = 4
BN = 400
NB = N // BN


def _sc_msg_pass(hist: bool):
  W = D
  mesh = plsc.VectorSubcoreMesh(core_axis_name="c", subcore_axis_name="s")

  def body(src_hbm, dst_hbm, et_hbm, tab_hbm, x_hbm, out_hbm,
           agg_sh, src_b, dst_b, et_b, rows_b, tab_v,
           sem_i, sem_g, sem_s):
    c = lax.axis_index("c")
    s = lax.axis_index("s")
    b = c
    lane = lax.iota(jnp.int32, 16)
    zero16 = jnp.zeros((16,), jnp.float32)

    if not hist:
      pltpu.sync_copy(tab_hbm.at[pl.ds(b * 24, 24)], tab_v)

    def rz(i, carry):
      for j in range(W // 16):
        for rv in (rows_b if hist else rows_b[:1]):
          rv[i, pl.ds(j * 16, 16)] = zero16
      return carry
    lax.fori_loop(0, CH, rz, 0)
    for m in range(ROWS_PT // CH):
      pltpu.sync_copy(rows_b[0], agg_sh.at[pl.ds(s * ROWS_PT + m * CH, CH)])

    def issue_idx(k, p, pd):
      base = b * E_PAD + s * EPT + k * CH
      if not hist:
        pltpu.async_copy(src_hbm.at[pl.ds(base, CH)], src_b[p], sem_i[p])
      pltpu.async_copy(dst_hbm.at[pl.ds(base, CH)], dst_b[pd], sem_i[p])
      pltpu.async_copy(et_hbm.at[pl.ds(base, CH)], et_b[p].at[pl.ds(0, CH)],
                       sem_i[p])

    def wait_idx(k, p, pd):
      base = b * E_PAD + s * EPT + k * CH
      if not hist:
        pltpu.make_async_copy(
            src_hbm.at[pl.ds(base, CH)], src_b[p], sem_i[p]).wait()
      pltpu.make_async_copy(
          dst_hbm.at[pl.ds(base, CH)], dst_b[pd], sem_i[p]).wait()
      pltpu.make_async_copy(
          et_hbm.at[pl.ds(base, CH)], et_b[p].at[pl.ds(0, CH)],
          sem_i[p]).wait()

    def wait_scatter(p, pd):
      pltpu.make_async_copy(
          rows_b[p], agg_sh.at[dst_b[pd]], sem_s[p]).wait()

    for p in range(NBUF):
      issue_idx(p, p, p)
    if not hist:
      for p in range(NBUF - 1):
        wait_idx(p, p, p)
        pltpu.async_copy(x_hbm.at[src_b[p]], rows_b[p], sem_g[p])
    plsc.subcore_barrier()

    def compute_chunk(p):
      def group_body(g, carry):
        ets16 = et_b[p][pl.ds(g * 4, 16)]
        for ii in range(4):
          et_i = ets16[ii]
          i = g * 4 + ii
          if hist:
            rows_b[p][i, pl.ds(0, 16)] = jnp.where(
                lane == et_i, jnp.float32(1.0), jnp.float32(0.0))
          else:
            for j in range(D // 16):
              rows_b[p][i, pl.ds(j * 16, 16)] = (
                  rows_b[p][i, pl.ds(j * 16, 16)]
                  * tab_v[et_i, pl.ds(j * 16, 16)])
        return carry
      lax.fori_loop(0, CH // 4, group_body, 0)

    def chunk_oct(k8, carry):
      for o in range(2 * NBUF):
        p = o % NBUF
        k = 2 * NBUF * k8 + o
        if hist:
          @pl.when(k >= NBUF)
          def _():
            wait_scatter(p, (o + NBUF) % (2 * NBUF))
          wait_idx(k, p, o)
        else:
          pltpu.make_async_copy(
              x_hbm.at[src_b[p]], rows_b[p], sem_g[p]).wait()
        compute_chunk(p)
        pltpu.async_copy(rows_b[p], agg_sh.at[dst_b[o]], sem_s[p],
                         add=True)
        if not hist:
          @pl.when(k + NBUF - 1 < NCHUNK)
          def _():
            q = (o + NBUF - 1) % NBUF
            qd = (o + NBUF - 1) % (2 * NBUF)
            wait_idx(k + NBUF - 1, q, qd)
            @pl.when(k >= 1)
            def _():
              wait_scatter(q, (o - 1) % (2 * NBUF))
            pltpu.async_copy(x_hbm.at[src_b[q]], rows_b[q], sem_g[q])

        @pl.when(k + NBUF < NCHUNK)
        def _():
          issue_idx(k + NBUF, p, (o + NBUF) % (2 * NBUF))
      return carry
    lax.fori_loop(0, NCHUNK // (2 * NBUF), chunk_oct, 0)

    for p in range(NBUF):
      wait_scatter(p, (NCHUNK - NBUF + p) % (2 * NBUF))
    plsc.subcore_barrier()

    pltpu.sync_copy(agg_sh.at[pl.ds(s * ROWS_PT, ROWS_PT)],
                    out_hbm.at[pl.ds(b * N_PAD + s * ROWS_PT, ROWS_PT)])

  kern = pl.kernel(
      body,
      out_type=jax.ShapeDtypeStruct((B * N_PAD, W), jnp.float32),
      mesh=mesh,
      scratch_types=[
          pltpu.VMEM_SHARED((N_PAD, W), jnp.float32),
          [pltpu.VMEM((CH,), jnp.int32) for _ in range(NBUF)],
          [pltpu.VMEM((CH,), jnp.int32) for _ in range(2 * NBUF)],
          [pltpu.VMEM((CH + 16,), jnp.int32) for _ in range(NBUF)],
          [pltpu.VMEM((CH, W), jnp.float32) for _ in range(NBUF)],
          pltpu.VMEM((8, D) if hist else (24, D), jnp.float32),
          [pltpu.SemaphoreType.DMA for _ in range(NBUF)],
          [pltpu.SemaphoreType.DMA for _ in range(NBUF)],
          [pltpu.SemaphoreType.DMA for _ in range(NBUF)],
      ],
  )

  def call(src, dst, et, tab, x):
    if hist:
      x = jnp.zeros((8, D), jnp.float32)
      tab24 = jnp.zeros((B * 24, D), jnp.float32)
    else:
      tab24 = jnp.concatenate(
          [tab, jnp.zeros((B, 24 - tab.shape[1], D), jnp.float32)],
          axis=1).reshape(B * 24, D)
    return kern(src.reshape(-1), dst.reshape(-1), et.reshape(-1), tab24, x)
  return call


_sc_pass_hist = _sc_msg_pass(True)
_sc_pass_gather = _sc_msg_pass(False)


def _k0_body(rel_ref, w1_ref, b1_ref, w2_ref, b2_ref, q_ref,
             qrel1_ref, rel2_ref):
  ra = rel_ref[...]
  dn = (((1,), (1,)), ((), ()))
  r1 = lax.dot_general(ra, w1_ref[...], dn) + b1_ref[...]
  r2 = lax.dot_general(ra, w2_ref[...], dn) + b2_ref[...]
  q = q_ref[...]
  qb = jnp.broadcast_to(q[:, None, :], (B, R, D)).reshape(B * R, D)
  qrel1_ref[...] = r1 * qb
  rel2_ref[...] = r2


def _k0(rel_flat, rpW1, rpb1, rpW2, rpb2, query):
  return pl.pallas_call(
      _k0_body,
      out_shape=[
          jax.ShapeDtypeStruct((B * R, D), jnp.float32),
          jax.ShapeDtypeStruct((B * R, D), jnp.float32),
      ],
  )(rel_flat, rpW1, rpb1, rpW2, rpb2, query)


def _dense_body(final, hist, x_ref, a_ref, tab_ref, w1_ref, w2_ref, lb_ref,
                lng_ref, lnb_ref, q_ref, o_ref):
  x = x_ref[0]
  dn = (((1,), (1,)), ((), ()))
  if hist:
    cnt = a_ref[0][:, :16]
    a = lax.dot_general(cnt, tab_ref[0], (((1,), (0,)), ((), ()))) + x
  else:
    a = a_ref[0]
  t = (lax.dot_general(x, w1_ref[...], dn)
       + lax.dot_general(a, w2_ref[...], dn) + lb_ref[...])
  mu = jnp.mean(t, axis=-1, keepdims=True)
  d = t - mu
  var = jnp.mean(d * d, axis=-1, keepdims=True)
  y = d * lax.rsqrt(var + 1e-5) * lng_ref[...] + lnb_ref[...]
  y = jnp.maximum(y, 0.0) + x
  if final:
    qb = jnp.broadcast_to(q_ref[0, 0:1, :], (BN, D))
    o_ref[0] = jnp.concatenate([y, qb], axis=-1)
  else:
    o_ref[0] = y


def _dense(final, hist, x, agg_pad, tab, w1, w2, lb, lng, lnb, query):
  od = 2 * D if final else D
  aw = D
  full = lambda shape: pl.BlockSpec(shape, lambda bb, nb: (0, 0))
  if tab is None:
    tab = jnp.zeros((B, 1, 8, D), jnp.float32)
  return pl.pallas_call(
      functools.partial(_dense_body, final, hist),
      grid=(B, NB),
      in_specs=[
          pl.BlockSpec((1, BN, D), lambda bb, nb: (bb, nb, 0)),
          pl.BlockSpec((1, BN, aw), lambda bb, nb: (bb, nb, 0)),
          pl.BlockSpec((1, 16, D), lambda bb, nb: (bb, 0, 0)),
          full((D, D)),
          full((D, D)),
          full((1, D)),
          full((1, D)),
          full((1, D)),
          pl.BlockSpec((1, 8, D), lambda bb, nb: (bb, 0, 0)),
      ],
      out_specs=pl.BlockSpec((1, BN, od), lambda bb, nb: (bb, nb, 0)),
      out_shape=jax.ShapeDtypeStruct((B, N, od), jnp.float32),
  )(x, agg_pad, tab.reshape(B, 16, D) if hist else jnp.zeros((B, 16, D), jnp.float32),
    w1, w2, lb, lng, lnb,
    jnp.broadcast_to(query[:, None, :], (B, 8, D)))


def kernel(relation_representations, rpW1, rpb1, lW1, lb1, lng1, lnb1,
           rpW2, rpb2, lW2, lb2, lng2, lnb2,
           h_index, r_index, edge_index, edge_type):
  rel = relation_representations.astype(jnp.float32)
  h_index = h_index.astype(jnp.int32)
  r_index = r_index.astype(jnp.int32)
  src = edge_index[0].astype(jnp.int32)
  dst = edge_index[1].astype(jnp.int32)
  et = edge_type.astype(jnp.int32)

  query = jnp.take_along_axis(rel, r_index[:, None, None], axis=1)[:, 0, :]

  qrel1, rel2p = _k0(rel.reshape(B * R, D), rpW1, rpb1.reshape(1, D),
                     rpW2, rpb2.reshape(1, D), query)
  tab2 = jnp.concatenate(
      [rel2p.reshape(B, R, D), jnp.ones((B, 1, D), jnp.float32)], axis=1)

  npad = E_PAD - E - 1
  iz = jnp.zeros((npad,), jnp.int32)
  dump = jnp.full((npad,), N, jnp.int32)
  et_b = jnp.broadcast_to(et[None], (B, E))
  etA = jnp.concatenate(
      [et_b, jnp.full((B, 1), R, jnp.int32),
       jnp.broadcast_to(iz[None], (B, npad))], axis=1)
  dst_pad = jnp.concatenate(
      [jnp.broadcast_to(dst[None], (B, E)), h_index[:, None],
       jnp.broadcast_to(dump[None], (B, npad))], axis=1)
  m1 = src[None, :] == h_index[:, None]
  dstA = jnp.concatenate(
      [jnp.where(m1, dst[None, :], N), jnp.full((B, 1), N, jnp.int32),
       jnp.broadcast_to(dump[None], (B, npad))], axis=1)
  boff = jnp.arange(B, dtype=jnp.int32)[:, None] * N
  srcB = jnp.concatenate(
      [src[None, :] + boff,
       B * N + 8 * jnp.arange(B, dtype=jnp.int32)[:, None],
       jnp.broadcast_to(iz[None], (B, npad))], axis=1)

  cnt1 = _sc_pass_hist(srcB, dstA, etA, None, None)
  cnt1 = cnt1.reshape(B, N_PAD, D)
  hoh = (jnp.arange(N, dtype=jnp.int32)[None, :] == h_index[:, None])
  x0 = jnp.where(hoh[:, :, None], query[:, None, :], 0.0)

  h1 = _dense(False, True, x0, cnt1, qrel1, lW1[:, :D], lW1[:, D:],
              lb1.reshape(1, D), lng1.reshape(1, D), lnb1.reshape(1, D),
              query)

  qpad = jnp.zeros((16, D), jnp.float32).at[jnp.arange(B) * 8].set(query)
  x1 = jnp.concatenate([h1.reshape(B * N, D), qpad], axis=0)
  agg2 = _sc_pass_gather(srcB, dst_pad, etA, tab2, x1)
  agg2 = agg2.reshape(B, N_PAD, D)

  return _dense(True, False, h1, agg2, None, lW2[:, :D], lW2[:, D:],
                lb2.reshape(1, D), lng2.reshape(1, D), lnb2.reshape(1, D),
                query)


# speedup vs baseline: 37.7923x; 1.0027x over previous
import functools

import jax
import jax.numpy as jnp
from jax import lax
from jax.experimental import pallas as pl
from jax.experimental.pallas import tpu as pltpu
from jax.experimental.pallas import tpu_sc as plsc

N = 10000
E = 160000
D = 128
R = 16
B = 2

NTILE = 16
N_PAD = 10240
ROWS_PT = N_PAD // NTILE
CH = 80
E_PAD = 163840
EPT = E_PAD // NTILE
NCHUNK = EPT // CH
NBUF = 4
BN = 400
NB = N // BN


def _sc_msg_pass(hist: bool):
  W = 32 if hist else D
  mesh = plsc.VectorSubcoreMesh(core_axis_name="c", subcore_axis_name="s")

  def body(src_hbm, dst_hbm, et_hbm, tab_hbm, x_hbm, out_hbm,
           agg_sh, src_b, dst_b, et_b, rows_b, tab_v,
           sem_i, sem_g, sem_s):
    c = lax.axis_index("c")
    s = lax.axis_index("s")
    b = c
    lane = lax.iota(jnp.int32, 16)
    zero16 = jnp.zeros((16,), jnp.float32)

    if not hist:
      pltpu.sync_copy(tab_hbm.at[pl.ds(b * 24, 24)], tab_v)

    def rz(i, carry):
      for j in range(W // 16):
        for rv in (rows_b if hist else rows_b[:1]):
          rv[i, pl.ds(j * 16, 16)] = zero16
      return carry
    lax.fori_loop(0, CH, rz, 0)
    for m in range(ROWS_PT // CH):
      pltpu.sync_copy(rows_b[0], agg_sh.at[pl.ds(s * ROWS_PT + m * CH, CH)])

    def issue_idx(k, p, pd):
      base = b * E_PAD + s * EPT + k * CH
      if not hist:
        pltpu.async_copy(src_hbm.at[pl.ds(base, CH)], src_b[p], sem_i[p])
      pltpu.async_copy(dst_hbm.at[pl.ds(base, CH)], dst_b[pd], sem_i[p])
      pltpu.async_copy(et_hbm.at[pl.ds(base, CH)], et_b[p].at[pl.ds(0, CH)],
                       sem_i[p])

    def wait_idx(k, p, pd):
      base = b * E_PAD + s * EPT + k * CH
      if not hist:
        pltpu.make_async_copy(
            src_hbm.at[pl.ds(base, CH)], src_b[p], sem_i[p]).wait()
      pltpu.make_async_copy(
          dst_hbm.at[pl.ds(base, CH)], dst_b[pd], sem_i[p]).wait()
      pltpu.make_async_copy(
          et_hbm.at[pl.ds(base, CH)], et_b[p].at[pl.ds(0, CH)],
          sem_i[p]).wait()

    def wait_scatter(p, pd):
      pltpu.make_async_copy(
          rows_b[p], agg_sh.at[dst_b[pd]], sem_s[p]).wait()

    for p in range(NBUF):
      issue_idx(p, p, p)
    if not hist:
      for p in range(NBUF - 1):
        wait_idx(p, p, p)
        pltpu.async_copy(x_hbm.at[src_b[p]], rows_b[p], sem_g[p])
    plsc.subcore_barrier()

    def compute_chunk(p):
      def group_body(g, carry):
        ets16 = et_b[p][pl.ds(g * 4, 16)]
        for ii in range(4):
          et_i = ets16[ii]
          i = g * 4 + ii
          if hist:
            rows_b[p][i, pl.ds(0, 16)] = jnp.where(
                lane == et_i, jnp.float32(1.0), jnp.float32(0.0))
          else:
            for j in range(D // 16):
              rows_b[p][i, pl.ds(j * 16, 16)] = (
                  rows_b[p][i, pl.ds(j * 16, 16)]
                  * tab_v[et_i, pl.ds(j * 16, 16)])
        return carry
      lax.fori_loop(0, CH // 4, group_body, 0)

    def chunk_oct(k8, carry):
      for o in range(2 * NBUF):
        p = o % NBUF
        k = 2 * NBUF * k8 + o
        if hist:
          @pl.when(k >= NBUF)
          def _():
            wait_scatter(p, (o + NBUF) % (2 * NBUF))
          wait_idx(k, p, o)
        else:
          pltpu.make_async_copy(
              x_hbm.at[src_b[p]], rows_b[p], sem_g[p]).wait()
        compute_chunk(p)
        pltpu.async_copy(rows_b[p], agg_sh.at[dst_b[o]], sem_s[p],
                         add=True)
        if not hist:
          @pl.when(k + NBUF - 1 < NCHUNK)
          def _():
            q = (o + NBUF - 1) % NBUF
            qd = (o + NBUF - 1) % (2 * NBUF)
            wait_idx(k + NBUF - 1, q, qd)
            @pl.when(k >= 1)
            def _():
              wait_scatter(q, (o - 1) % (2 * NBUF))
            pltpu.async_copy(x_hbm.at[src_b[q]], rows_b[q], sem_g[q])

        @pl.when(k + NBUF < NCHUNK)
        def _():
          issue_idx(k + NBUF, p, (o + NBUF) % (2 * NBUF))
      return carry
    lax.fori_loop(0, NCHUNK // (2 * NBUF), chunk_oct, 0)

    for p in range(NBUF):
      wait_scatter(p, (NCHUNK - NBUF + p) % (2 * NBUF))
    plsc.subcore_barrier()

    pltpu.sync_copy(agg_sh.at[pl.ds(s * ROWS_PT, ROWS_PT)],
                    out_hbm.at[pl.ds(b * N_PAD + s * ROWS_PT, ROWS_PT)])

  kern = pl.kernel(
      body,
      out_type=jax.ShapeDtypeStruct((B * N_PAD, W), jnp.float32),
      mesh=mesh,
      compiler_params=(pltpu.CompilerParams(use_tc_tiling_on_sc=False)
                       if hist else None),
      scratch_types=[
          pltpu.VMEM_SHARED((N_PAD, W), jnp.float32),
          [pltpu.VMEM((CH,), jnp.int32) for _ in range(NBUF)],
          [pltpu.VMEM((CH,), jnp.int32) for _ in range(2 * NBUF)],
          [pltpu.VMEM((CH + 16,), jnp.int32) for _ in range(NBUF)],
          [pltpu.VMEM((CH, W), jnp.float32) for _ in range(NBUF)],
          pltpu.VMEM((8, D) if hist else (24, D), jnp.float32),
          [pltpu.SemaphoreType.DMA for _ in range(NBUF)],
          [pltpu.SemaphoreType.DMA for _ in range(NBUF)],
          [pltpu.SemaphoreType.DMA for _ in range(NBUF)],
      ],
  )

  def call(src, dst, et, tab, x):
    if hist:
      x = jnp.zeros((8, D), jnp.float32)
      tab24 = jnp.zeros((B * 24, D), jnp.float32)
    else:
      tab24 = jnp.concatenate(
          [tab, jnp.zeros((B, 24 - tab.shape[1], D), jnp.float32)],
          axis=1).reshape(B * 24, D)
    return kern(src.reshape(-1), dst.reshape(-1), et.reshape(-1), tab24, x)
  return call


_sc_pass_hist = _sc_msg_pass(True)
_sc_pass_gather = _sc_msg_pass(False)


def _k0_body(rel_ref, w1_ref, b1_ref, w2_ref, b2_ref, q_ref,
             qrel1_ref, rel2_ref):
  ra = rel_ref[...]
  dn = (((1,), (1,)), ((), ()))
  r1 = lax.dot_general(ra, w1_ref[...], dn) + b1_ref[...]
  r2 = lax.dot_general(ra, w2_ref[...], dn) + b2_ref[...]
  q = q_ref[...]
  qb = jnp.broadcast_to(q[:, None, :], (B, R, D)).reshape(B * R, D)
  qrel1_ref[...] = r1 * qb
  rel2_ref[...] = r2


def _k0(rel_flat, rpW1, rpb1, rpW2, rpb2, query):
  return pl.pallas_call(
      _k0_body,
      out_shape=[
          jax.ShapeDtypeStruct((B * R, D), jnp.float32),
          jax.ShapeDtypeStruct((B * R, D), jnp.float32),
      ],
  )(rel_flat, rpW1, rpb1, rpW2, rpb2, query)


def _dense_body(final, hist, x_ref, a_ref, tab_ref, w1_ref, w2_ref, lb_ref,
                lng_ref, lnb_ref, q_ref, o_ref):
  x = x_ref[0]
  dn = (((1,), (1,)), ((), ()))
  if hist:
    cnt = a_ref[0][:, :16]
    a = lax.dot_general(cnt, tab_ref[0], (((1,), (0,)), ((), ()))) + x
  else:
    a = a_ref[0]
  t = (lax.dot_general(x, w1_ref[...], dn)
       + lax.dot_general(a, w2_ref[...], dn) + lb_ref[...])
  mu = jnp.mean(t, axis=-1, keepdims=True)
  d = t - mu
  var = jnp.mean(d * d, axis=-1, keepdims=True)
  y = d * lax.rsqrt(var + 1e-5) * lng_ref[...] + lnb_ref[...]
  y = jnp.maximum(y, 0.0) + x
  if final:
    qb = jnp.broadcast_to(q_ref[0, 0:1, :], (BN, D))
    o_ref[0] = jnp.concatenate([y, qb], axis=-1)
  else:
    o_ref[0] = y


def _dense(final, hist, x, agg_pad, tab, w1, w2, lb, lng, lnb, query):
  od = 2 * D if final else D
  aw = 32 if hist else D
  full = lambda shape: pl.BlockSpec(shape, lambda bb, nb: (0, 0))
  if tab is None:
    tab = jnp.zeros((B, 1, 8, D), jnp.float32)
  return pl.pallas_call(
      functools.partial(_dense_body, final, hist),
      grid=(B, NB),
      in_specs=[
          pl.BlockSpec((1, BN, D), lambda bb, nb: (bb, nb, 0)),
          pl.BlockSpec((1, BN, aw), lambda bb, nb: (bb, nb, 0)),
          pl.BlockSpec((1, 16, D), lambda bb, nb: (bb, 0, 0)),
          full((D, D)),
          full((D, D)),
          full((1, D)),
          full((1, D)),
          full((1, D)),
          pl.BlockSpec((1, 8, D), lambda bb, nb: (bb, 0, 0)),
      ],
      out_specs=pl.BlockSpec((1, BN, od), lambda bb, nb: (bb, nb, 0)),
      out_shape=jax.ShapeDtypeStruct((B, N, od), jnp.float32),
  )(x, agg_pad, tab.reshape(B, 16, D) if hist else jnp.zeros((B, 16, D), jnp.float32),
    w1, w2, lb, lng, lnb,
    jnp.broadcast_to(query[:, None, :], (B, 8, D)))


def kernel(relation_representations, rpW1, rpb1, lW1, lb1, lng1, lnb1,
           rpW2, rpb2, lW2, lb2, lng2, lnb2,
           h_index, r_index, edge_index, edge_type):
  rel = relation_representations.astype(jnp.float32)
  h_index = h_index.astype(jnp.int32)
  r_index = r_index.astype(jnp.int32)
  src = edge_index[0].astype(jnp.int32)
  dst = edge_index[1].astype(jnp.int32)
  et = edge_type.astype(jnp.int32)

  query = jnp.take_along_axis(rel, r_index[:, None, None], axis=1)[:, 0, :]

  qrel1, rel2p = _k0(rel.reshape(B * R, D), rpW1, rpb1.reshape(1, D),
                     rpW2, rpb2.reshape(1, D), query)
  tab2 = jnp.concatenate(
      [rel2p.reshape(B, R, D), jnp.ones((B, 1, D), jnp.float32)], axis=1)

  npad = E_PAD - E - 1
  iz = jnp.zeros((npad,), jnp.int32)
  dump = jnp.full((npad,), N, jnp.int32)
  et_b = jnp.broadcast_to(et[None], (B, E))
  etA = jnp.concatenate(
      [et_b, jnp.full((B, 1), R, jnp.int32),
       jnp.broadcast_to(iz[None], (B, npad))], axis=1)
  dst_pad = jnp.concatenate(
      [jnp.broadcast_to(dst[None], (B, E)), h_index[:, None],
       jnp.broadcast_to(dump[None], (B, npad))], axis=1)
  m1 = src[None, :] == h_index[:, None]
  dstA = jnp.concatenate(
      [jnp.where(m1, dst[None, :], N), jnp.full((B, 1), N, jnp.int32),
       jnp.broadcast_to(dump[None], (B, npad))], axis=1)
  boff = jnp.arange(B, dtype=jnp.int32)[:, None] * N
  srcB = jnp.concatenate(
      [src[None, :] + boff,
       B * N + 8 * jnp.arange(B, dtype=jnp.int32)[:, None],
       jnp.broadcast_to(iz[None], (B, npad))], axis=1)

  cnt1 = _sc_pass_hist(srcB, dstA, etA, None, None)
  cnt1 = cnt1.reshape(B, N_PAD, 32)
  hoh = (jnp.arange(N, dtype=jnp.int32)[None, :] == h_index[:, None])
  x0 = jnp.where(hoh[:, :, None], query[:, None, :], 0.0)

  h1 = _dense(False, True, x0, cnt1, qrel1, lW1[:, :D], lW1[:, D:],
              lb1.reshape(1, D), lng1.reshape(1, D), lnb1.reshape(1, D),
              query)

  qpad = jnp.zeros((16, D), jnp.float32).at[jnp.arange(B) * 8].set(query)
  x1 = jnp.concatenate([h1.reshape(B * N, D), qpad], axis=0)
  agg2 = _sc_pass_gather(srcB, dst_pad, etA, tab2, x1)
  agg2 = agg2.reshape(B, N_PAD, D)

  return _dense(True, False, h1, agg2, None, lW2[:, :D], lW2[:, D:],
                lb2.reshape(1, D), lng2.reshape(1, D), lnb2.reshape(1, D),
                query)


# speedup vs baseline: 37.8492x; 1.0015x over previous
import functools

import jax
import jax.numpy as jnp
from jax import lax
from jax.experimental import pallas as pl
from jax.experimental.pallas import tpu as pltpu
from jax.experimental.pallas import tpu_sc as plsc

N = 10000
E = 160000
D = 128
R = 16
B = 2

NTILE = 16
N_PAD = 10240
ROWS_PT = N_PAD // NTILE
CH = 80
E_PAD = 163840
EPT = E_PAD // NTILE
NCHUNK = EPT // CH
NBUF = 4
BN = 400
NB = N // BN


def _sc_msg_pass_r4(hist: bool):
  W = 32 if hist else D
  mesh = plsc.VectorSubcoreMesh(core_axis_name="c", subcore_axis_name="s")

  def body(src_hbm, dst_hbm, et_hbm, tab_hbm, x_hbm, out_hbm,
           agg_sh, src_b, dst_b, et_b, rows_b, tab_v,
           sem_i, sem_g, sem_s):
    c = lax.axis_index("c")
    s = lax.axis_index("s")
    b = c
    lane = lax.iota(jnp.int32, 16)
    zero16 = jnp.zeros((16,), jnp.float32)

    if not hist:
      pltpu.sync_copy(tab_hbm.at[pl.ds(b * 24, 24)], tab_v)

    def rz(i, carry):
      for j in range(W // 16):
        for rv in (rows_b if hist else rows_b[:1]):
          rv[i, pl.ds(j * 16, 16)] = zero16
      return carry
    lax.fori_loop(0, CH, rz, 0)
    for m in range(ROWS_PT // CH):
      pltpu.sync_copy(rows_b[0], agg_sh.at[pl.ds(s * ROWS_PT + m * CH, CH)])

    def issue_idx(k, p, pd):
      base = b * E_PAD + s * EPT + k * CH
      if not hist:
        pltpu.async_copy(src_hbm.at[pl.ds(base, CH)], src_b[p], sem_i[p])
      pltpu.async_copy(dst_hbm.at[pl.ds(base, CH)], dst_b[pd], sem_i[p])
      pltpu.async_copy(et_hbm.at[pl.ds(base, CH)], et_b[p].at[pl.ds(0, CH)],
                       sem_i[p])

    def wait_idx(k, p, pd):
      base = b * E_PAD + s * EPT + k * CH
      if not hist:
        pltpu.make_async_copy(
            src_hbm.at[pl.ds(base, CH)], src_b[p], sem_i[p]).wait()
      pltpu.make_async_copy(
          dst_hbm.at[pl.ds(base, CH)], dst_b[pd], sem_i[p]).wait()
      pltpu.make_async_copy(
          et_hbm.at[pl.ds(base, CH)], et_b[p].at[pl.ds(0, CH)],
          sem_i[p]).wait()

    def wait_scatter(p, pd):
      pltpu.make_async_copy(
          rows_b[p], agg_sh.at[dst_b[pd]], sem_s[p]).wait()

    for p in range(NBUF):
      issue_idx(p, p, p)
    if not hist:
      for p in range(NBUF - 1):
        wait_idx(p, p, p)
        pltpu.async_copy(x_hbm.at[src_b[p]], rows_b[p], sem_g[p])
    plsc.subcore_barrier()

    def compute_chunk(p):
      def group_body(g, carry):
        ets16 = et_b[p][pl.ds(g * 4, 16)]
        for ii in range(4):
          et_i = ets16[ii]
          i = g * 4 + ii
          if hist:
            rows_b[p][i, pl.ds(0, 16)] = jnp.where(
                lane == et_i, jnp.float32(1.0), jnp.float32(0.0))
          else:
            for j in range(D // 16):
              rows_b[p][i, pl.ds(j * 16, 16)] = (
                  rows_b[p][i, pl.ds(j * 16, 16)]
                  * tab_v[et_i, pl.ds(j * 16, 16)])
        return carry
      lax.fori_loop(0, CH // 4, group_body, 0)

    def chunk_oct(k8, carry):
      for o in range(2 * NBUF):
        p = o % NBUF
        k = 2 * NBUF * k8 + o
        if hist:
          @pl.when(k >= NBUF)
          def _():
            wait_scatter(p, (o + NBUF) % (2 * NBUF))
          wait_idx(k, p, o)
        else:
          pltpu.make_async_copy(
              x_hbm.at[src_b[p]], rows_b[p], sem_g[p]).wait()
        compute_chunk(p)
        pltpu.async_copy(rows_b[p], agg_sh.at[dst_b[o]], sem_s[p],
                         add=True)
        if not hist:
          @pl.when(k + NBUF - 1 < NCHUNK)
          def _():
            q = (o + NBUF - 1) % NBUF
            qd = (o + NBUF - 1) % (2 * NBUF)
            wait_idx(k + NBUF - 1, q, qd)
            @pl.when(k >= 1)
            def _():
              wait_scatter(q, (o - 1) % (2 * NBUF))
            pltpu.async_copy(x_hbm.at[src_b[q]], rows_b[q], sem_g[q])

        @pl.when(k + NBUF < NCHUNK)
        def _():
          issue_idx(k + NBUF, p, (o + NBUF) % (2 * NBUF))
      return carry
    lax.fori_loop(0, NCHUNK // (2 * NBUF), chunk_oct, 0)

    for p in range(NBUF):
      wait_scatter(p, (NCHUNK - NBUF + p) % (2 * NBUF))
    plsc.subcore_barrier()

    pltpu.sync_copy(agg_sh.at[pl.ds(s * ROWS_PT, ROWS_PT)],
                    out_hbm.at[pl.ds(b * N_PAD + s * ROWS_PT, ROWS_PT)])

  kern = pl.kernel(
      body,
      out_type=jax.ShapeDtypeStruct((B * N_PAD, W), jnp.float32),
      mesh=mesh,
      compiler_params=(pltpu.CompilerParams(use_tc_tiling_on_sc=False)
                       if hist else None),
      scratch_types=[
          pltpu.VMEM_SHARED((N_PAD, W), jnp.float32),
          [pltpu.VMEM((CH,), jnp.int32) for _ in range(NBUF)],
          [pltpu.VMEM((CH,), jnp.int32) for _ in range(2 * NBUF)],
          [pltpu.VMEM((CH + 16,), jnp.int32) for _ in range(NBUF)],
          [pltpu.VMEM((CH, W), jnp.float32) for _ in range(NBUF)],
          pltpu.VMEM((8, D) if hist else (24, D), jnp.float32),
          [pltpu.SemaphoreType.DMA for _ in range(NBUF)],
          [pltpu.SemaphoreType.DMA for _ in range(NBUF)],
          [pltpu.SemaphoreType.DMA for _ in range(NBUF)],
      ],
  )

  def call(src, dst, et, tab, x):
    if hist:
      x = jnp.zeros((8, D), jnp.float32)
      tab24 = jnp.zeros((B * 24, D), jnp.float32)
    else:
      tab24 = jnp.concatenate(
          [tab, jnp.zeros((B, 24 - tab.shape[1], D), jnp.float32)],
          axis=1).reshape(B * 24, D)
    return kern(src.reshape(-1), dst.reshape(-1), et.reshape(-1), tab24, x)
  return call


def _sc_msg_pass(hist: bool):
  W = 32 if hist else D
  mesh = plsc.VectorSubcoreMesh(core_axis_name="c", subcore_axis_name="s")

  NB_ = 4 if hist else 2
  CH_ = CH if hist else 64
  NCHUNK_ = EPT // CH_

  def body(src_hbm, dst_hbm, et_hbm, tab_hbm, x_hbm, out_hbm,
           agg_sh, comb_v, dst2_v, srcv_b, rows_b, tab_v,
           sem_g, sem_s):
    c = lax.axis_index("c")
    s = lax.axis_index("s")
    b = c
    lane = lax.iota(jnp.int32, 16)
    zero16 = jnp.zeros((16,), jnp.float32)

    if not hist:
      pltpu.sync_copy(tab_hbm.at[pl.ds(b * 24, 24)], tab_v)

    pltpu.sync_copy(dst_hbm.at[pl.ds((b * NTILE + s) * NCHUNK_, NCHUNK_)],
                    dst2_v)
    if hist:
      pltpu.sync_copy(et_hbm.at[pl.ds(b * E_PAD + s * EPT, EPT)],
                      comb_v.at[pl.ds(0, EPT)])
    else:
      pltpu.sync_copy(src_hbm.at[pl.ds(b * E_PAD + s * EPT, EPT)],
                      comb_v.at[pl.ds(0, EPT)])

    def rz(i, carry):
      for j in range(W // 16):
        for rv in (rows_b if hist else rows_b[:1]):
          rv[i, pl.ds(j * 16, 16)] = zero16
      return carry
    lax.fori_loop(0, CH_, rz, 0)
    for m in range(ROWS_PT // CH_):
      pltpu.sync_copy(rows_b[0], agg_sh.at[pl.ds(s * ROWS_PT + m * CH_, CH_)])

    def unpack_src(k, p):
      def ug(g, carry):
        w = comb_v[pl.ds(k * CH_ + g * 16, 16)]
        srcv_b[p][pl.ds(g * 16, 16)] = w & jnp.int32(0xFFFFF)
        return carry
      lax.fori_loop(0, CH_ // 16, ug, 0)

    def wait_scatter(p, k):
      pltpu.make_async_copy(
          rows_b[p], agg_sh.at[dst2_v.at[k]], sem_s[p]).wait()

    if not hist:
      unpack_src(0, 0)
      pltpu.async_copy(x_hbm.at[srcv_b[0]], rows_b[0], sem_g[0])
    plsc.subcore_barrier()

    def compute_chunk(k, p):
      def group_body(g, carry):
        ets16 = comb_v[pl.ds(k * CH_ + g * 4, 16)]
        if not hist:
          ets16 = lax.shift_right_logical(ets16, 20)
        for ii in range(4):
          et_i = ets16[ii]
          i = g * 4 + ii
          if hist:
            rows_b[p][i, pl.ds(0, 16)] = jnp.where(
                lane == et_i, jnp.float32(1.0), jnp.float32(0.0))
          else:
            for j in range(D // 16):
              rows_b[p][i, pl.ds(j * 16, 16)] = (
                  rows_b[p][i, pl.ds(j * 16, 16)]
                  * tab_v[et_i, pl.ds(j * 16, 16)])
        return carry
      lax.fori_loop(0, CH_ // 4, group_body, 0)

    def chunk_grp(kg, carry):
      for p in range(NB_):
        k = NB_ * kg + p
        if hist:
          @pl.when(k >= NB_)
          def _():
            wait_scatter(p, k - NB_)
        else:
          pltpu.make_async_copy(
              x_hbm.at[srcv_b[p]], rows_b[p], sem_g[p]).wait()
        compute_chunk(k, p)
        pltpu.async_copy(rows_b[p], agg_sh.at[dst2_v.at[k]], sem_s[p],
                         add=True)
        if not hist:
          @pl.when(k + 1 < NCHUNK_)
          def _():
            q = (p + 1) % NB_
            unpack_src(k + 1, q)
            @pl.when(k >= 1)
            def _():
              wait_scatter(q, k - 1)
            pltpu.async_copy(x_hbm.at[srcv_b[q]], rows_b[q], sem_g[q])
      return carry
    lax.fori_loop(0, NCHUNK_ // NB_, chunk_grp, 0)

    for p in range(NB_):
      wait_scatter(p, NCHUNK_ - NB_ + p)
    plsc.subcore_barrier()

    pltpu.sync_copy(agg_sh.at[pl.ds(s * ROWS_PT, ROWS_PT)],
                    out_hbm.at[pl.ds(b * N_PAD + s * ROWS_PT, ROWS_PT)])

  kern = pl.kernel(
      body,
      out_type=jax.ShapeDtypeStruct((B * N_PAD, W), jnp.float32),
      mesh=mesh,
      compiler_params=(pltpu.CompilerParams(use_tc_tiling_on_sc=False)
                       if hist else None),
      scratch_types=[
          pltpu.VMEM_SHARED((N_PAD, W), jnp.float32),
          pltpu.VMEM((EPT + 16,), jnp.int32),
          pltpu.VMEM((NCHUNK_, CH_), jnp.int32),
          [pltpu.VMEM((8,) if hist else (CH_,), jnp.int32)
           for _ in range(NB_)],
          [pltpu.VMEM((CH_, W), jnp.float32) for _ in range(NB_)],
          pltpu.VMEM((8, D) if hist else (24, D), jnp.float32),
          [pltpu.SemaphoreType.DMA for _ in range(NB_)],
          [pltpu.SemaphoreType.DMA for _ in range(NB_)],
      ],
  )

  def call(src, dst, et, tab, x):
    dst2 = dst.reshape(B * NTILE * NCHUNK_, CH_)
    if hist:
      x = jnp.zeros((8, D), jnp.float32)
      tab24 = jnp.zeros((8, D), jnp.float32)
      srcin = jnp.zeros((8,), jnp.int32)
      etin = et.reshape(-1)
    else:
      tab24 = jnp.concatenate(
          [tab, jnp.zeros((B, 24 - tab.shape[1], D), jnp.float32)],
          axis=1).reshape(B * 24, D)
      srcin = (src | (et << 20)).reshape(-1)
      etin = jnp.zeros((8,), jnp.int32)
    return kern(srcin, dst2, etin, tab24, x)
  return call


_sc_pass_hist = _sc_msg_pass(True)
_sc_pass_gather = _sc_msg_pass_r4(False)


def _k0_body(rel_ref, w1_ref, b1_ref, w2_ref, b2_ref, q_ref,
             qrel1_ref, rel2_ref):
  ra = rel_ref[...]
  dn = (((1,), (1,)), ((), ()))
  r1 = lax.dot_general(ra, w1_ref[...], dn) + b1_ref[...]
  r2 = lax.dot_general(ra, w2_ref[...], dn) + b2_ref[...]
  q = q_ref[...]
  qb = jnp.broadcast_to(q[:, None, :], (B, R, D)).reshape(B * R, D)
  qrel1_ref[...] = r1 * qb
  rel2_ref[...] = r2


def _k0(rel_flat, rpW1, rpb1, rpW2, rpb2, query):
  return pl.pallas_call(
      _k0_body,
      out_shape=[
          jax.ShapeDtypeStruct((B * R, D), jnp.float32),
          jax.ShapeDtypeStruct((B * R, D), jnp.float32),
      ],
  )(rel_flat, rpW1, rpb1, rpW2, rpb2, query)


def _dense_body(final, hist, x_ref, a_ref, tab_ref, w1_ref, w2_ref, lb_ref,
                lng_ref, lnb_ref, q_ref, o_ref):
  x = x_ref[0]
  dn = (((1,), (1,)), ((), ()))
  if hist:
    cnt = a_ref[0][:, :16]
    a = lax.dot_general(cnt, tab_ref[0], (((1,), (0,)), ((), ()))) + x
  else:
    a = a_ref[0]
  t = (lax.dot_general(x, w1_ref[...], dn)
       + lax.dot_general(a, w2_ref[...], dn) + lb_ref[...])
  mu = jnp.mean(t, axis=-1, keepdims=True)
  d = t - mu
  var = jnp.mean(d * d, axis=-1, keepdims=True)
  y = d * lax.rsqrt(var + 1e-5) * lng_ref[...] + lnb_ref[...]
  y = jnp.maximum(y, 0.0) + x
  if final:
    qb = jnp.broadcast_to(q_ref[0, 0:1, :], (BN, D))
    o_ref[0] = jnp.concatenate([y, qb], axis=-1)
  else:
    o_ref[0] = y


def _dense(final, hist, x, agg_pad, tab, w1, w2, lb, lng, lnb, query):
  od = 2 * D if final else D
  aw = 32 if hist else D
  full = lambda shape: pl.BlockSpec(shape, lambda bb, nb: (0, 0))
  if tab is None:
    tab = jnp.zeros((B, 1, 8, D), jnp.float32)
  return pl.pallas_call(
      functools.partial(_dense_body, final, hist),
      grid=(B, NB),
      in_specs=[
          pl.BlockSpec((1, BN, D), lambda bb, nb: (bb, nb, 0)),
          pl.BlockSpec((1, BN, aw), lambda bb, nb: (bb, nb, 0)),
          pl.BlockSpec((1, 16, D), lambda bb, nb: (bb, 0, 0)),
          full((D, D)),
          full((D, D)),
          full((1, D)),
          full((1, D)),
          full((1, D)),
          pl.BlockSpec((1, 8, D), lambda bb, nb: (bb, 0, 0)),
      ],
      out_specs=pl.BlockSpec((1, BN, od), lambda bb, nb: (bb, nb, 0)),
      out_shape=jax.ShapeDtypeStruct((B, N, od), jnp.float32),
  )(x, agg_pad, tab.reshape(B, 16, D) if hist else jnp.zeros((B, 16, D), jnp.float32),
    w1, w2, lb, lng, lnb,
    jnp.broadcast_to(query[:, None, :], (B, 8, D)))


def kernel(relation_representations, rpW1, rpb1, lW1, lb1, lng1, lnb1,
           rpW2, rpb2, lW2, lb2, lng2, lnb2,
           h_index, r_index, edge_index, edge_type):
  rel = relation_representations.astype(jnp.float32)
  h_index = h_index.astype(jnp.int32)
  r_index = r_index.astype(jnp.int32)
  src = edge_index[0].astype(jnp.int32)
  dst = edge_index[1].astype(jnp.int32)
  et = edge_type.astype(jnp.int32)

  query = jnp.take_along_axis(rel, r_index[:, None, None], axis=1)[:, 0, :]

  qrel1, rel2p = _k0(rel.reshape(B * R, D), rpW1, rpb1.reshape(1, D),
                     rpW2, rpb2.reshape(1, D), query)
  tab2 = jnp.concatenate(
      [rel2p.reshape(B, R, D), jnp.ones((B, 1, D), jnp.float32)], axis=1)

  npad = E_PAD - E - 1
  iz = jnp.zeros((npad,), jnp.int32)
  dump = jnp.full((npad,), N, jnp.int32)
  et_b = jnp.broadcast_to(et[None], (B, E))
  etA = jnp.concatenate(
      [et_b, jnp.full((B, 1), R, jnp.int32),
       jnp.broadcast_to(iz[None], (B, npad))], axis=1)
  dst_pad = jnp.concatenate(
      [jnp.broadcast_to(dst[None], (B, E)), h_index[:, None],
       jnp.broadcast_to(dump[None], (B, npad))], axis=1)
  m1 = src[None, :] == h_index[:, None]
  dstA = jnp.concatenate(
      [jnp.where(m1, dst[None, :], N), jnp.full((B, 1), N, jnp.int32),
       jnp.broadcast_to(dump[None], (B, npad))], axis=1)
  boff = jnp.arange(B, dtype=jnp.int32)[:, None] * N
  srcB = jnp.concatenate(
      [src[None, :] + boff,
       B * N + 8 * jnp.arange(B, dtype=jnp.int32)[:, None],
       jnp.broadcast_to(iz[None], (B, npad))], axis=1)

  cnt1 = _sc_pass_hist(srcB, dstA, etA, None, None)
  cnt1 = cnt1.reshape(B, N_PAD, 32)
  hoh = (jnp.arange(N, dtype=jnp.int32)[None, :] == h_index[:, None])
  x0 = jnp.where(hoh[:, :, None], query[:, None, :], 0.0)

  h1 = _dense(False, True, x0, cnt1, qrel1, lW1[:, :D], lW1[:, D:],
              lb1.reshape(1, D), lng1.reshape(1, D), lnb1.reshape(1, D),
              query)

  qpad = jnp.zeros((16, D), jnp.float32).at[jnp.arange(B) * 8].set(query)
  x1 = jnp.concatenate([h1.reshape(B * N, D), qpad], axis=0)
  agg2 = _sc_pass_gather(srcB, dst_pad, etA, tab2, x1)
  agg2 = agg2.reshape(B, N_PAD, D)

  return _dense(True, False, h1, agg2, None, lW2[:, :D], lW2[:, D:],
                lb2.reshape(1, D), lng2.reshape(1, D), lnb2.reshape(1, D),
                query)
